# Initial kernel scaffold; baseline (speedup 1.0000x reference)
import functools

import jax
import jax.numpy as jnp
from jax import lax
from jax.experimental import pallas as pl
from jax.experimental.pallas import tpu as pltpu
from jax.experimental.pallas import tpu_sc as plsc

N = 10000
E = 160000
D_IN = 256
D_HID = 256
D_OUT = 128

NC = 2
NS = 16
K = 128
EPAD = 163840
RPT = N // NS
DEGW = 16

_mesh = plsc.VectorSubcoreMesh(core_axis_name="c", subcore_axis_name="s")


@functools.partial(
    pl.kernel,
    out_type=jax.ShapeDtypeStruct((NC, N, DEGW), jnp.float32),
    mesh=_mesh,
    scratch_types=[
        pltpu.VMEM((K,), jnp.int32),
        pltpu.VMEM((K, DEGW), jnp.float32),
        pltpu.VMEM_SHARED((N + NS, DEGW), jnp.float32),
    ],
)
def _sc_degree(dst_hbm, const_hbm, out_hbm, idxd, ones_v, acc):
    c = lax.axis_index("c")
    s = lax.axis_index("s")
    rbase = s * RPT
    pltpu.sync_copy(const_hbm.at[c, pl.ds(rbase, RPT)], acc.at[pl.ds(rbase, RPT)])
    pltpu.sync_copy(const_hbm.at[0, pl.ds(0, K)], ones_v)
    plsc.subcore_barrier()

    ept = EPAD // (NC * NS)
    ebase = (c * NS + s) * ept

    def body(i, carry):
        eoff = ebase + i * K
        pltpu.sync_copy(dst_hbm.at[pl.ds(eoff, K)], idxd)
        pltpu.sync_copy(ones_v, acc.at[idxd], add=True)
        return carry

    lax.fori_loop(0, ept // K, body, 0)
    plsc.subcore_barrier()
    pltpu.sync_copy(acc.at[pl.ds(rbase, RPT)], out_hbm.at[c, pl.ds(rbase, RPT)])


def _make_sc_agg(W):
    @functools.partial(
        pl.kernel,
        out_type=jax.ShapeDtypeStruct((NC, N, W), jnp.float32),
        mesh=_mesh,
        scratch_types=[
            pltpu.VMEM((K,), jnp.int32),
            pltpu.VMEM((K,), jnp.int32),
            pltpu.VMEM((K,), jnp.int32),
            pltpu.VMEM((K, W), jnp.float32),
            pltpu.VMEM_SHARED((N + NS, W), jnp.float32),
            pltpu.SemaphoreType.DMA,
        ],
    )
    def _sc_agg(hp_hbm, src_hbm, dst_hbm, out_hbm, idxs, idxs2, idxd, rows, acc, sem):
        c = lax.axis_index("c")
        s = lax.axis_index("s")
        rbase = s * RPT
        bias = c * N
        pltpu.sync_copy(hp_hbm.at[pl.ds(bias + rbase, RPT)], acc.at[pl.ds(rbase, RPT)])
        plsc.subcore_barrier()

        ept = EPAD // NS
        ebase = s * ept

        def body(i, carry):
            eoff = ebase + i * K
            pltpu.sync_copy(src_hbm.at[pl.ds(eoff, K)], idxs)
            pltpu.sync_copy(dst_hbm.at[pl.ds(eoff, K)], idxd)
            for j in range(K // 16):
                sl = pl.ds(j * 16, 16)
                idxs2[sl] = idxs[sl] + bias
            pltpu.async_copy(hp_hbm.at[idxs2], rows, sem).wait()
            pltpu.sync_copy(rows, acc.at[idxd], add=True)
            return carry

        lax.fori_loop(0, ept // K, body, 0)
        plsc.subcore_barrier()
        pltpu.sync_copy(acc.at[pl.ds(rbase, RPT)], out_hbm.at[c, pl.ds(rbase, RPT)])

    return _sc_agg


_sc_agg_128 = _make_sc_agg(D_HID // 2)
_sc_agg_64 = _make_sc_agg(D_OUT // 2)


BR = 1000


def _dinv_of(degp):
    return lax.rsqrt(degp[0, :, 0] + degp[1, :, 0])


def _tc1_body(x_ref, w_ref, degp_ref, hp_ref):
    dinv = _dinv_of(degp_ref[...])
    h = jnp.dot(x_ref[...], w_ref[...], preferred_element_type=jnp.float32)
    hp = h * dinv[:, None]
    hp_ref[0] = hp[:, : D_HID // 2]
    hp_ref[1] = hp[:, D_HID // 2 :]


def _tc2_body(agg_ref, degp_ref, b1_ref, w2_ref, hp2_ref):
    a = agg_ref[...]
    dinv = _dinv_of(degp_ref[...])
    h1 = jnp.concatenate([a[0], a[1]], axis=1) * dinv[:, None] + b1_ref[...]
    h1 = jnp.maximum(h1, 0.0)
    h2 = jnp.dot(h1, w2_ref[...], preferred_element_type=jnp.float32)
    h2p = h2 * dinv[:, None]
    hp2_ref[0] = h2p[:, : D_OUT // 2]
    hp2_ref[1] = h2p[:, D_OUT // 2 :]


def _tc3_body(agg_ref, degp_ref, b2_ref, out_ref):
    a = agg_ref[...]
    dinv = _dinv_of(degp_ref[...])
    out = jnp.concatenate([a[0], a[1]], axis=1) * dinv[:, None] + b2_ref[...]
    out_ref[...] = jnp.maximum(out, 0.0)


_degp_spec = pl.BlockSpec((NC, BR, DEGW), lambda i: (0, i, 0))


def _tc_linear1(x, W1, degp):
    return pl.pallas_call(
        _tc1_body,
        grid=(N // BR,),
        in_specs=[
            pl.BlockSpec((BR, D_IN), lambda i: (i, 0)),
            pl.BlockSpec((D_IN, D_HID), lambda i: (0, 0)),
            _degp_spec,
        ],
        out_specs=pl.BlockSpec((NC, BR, D_HID // 2), lambda i: (0, i, 0)),
        out_shape=jax.ShapeDtypeStruct((NC, N, D_HID // 2), jnp.float32),
    )(x, W1, degp)


def _tc_linear2(agg1, degp, b1, W2):
    return pl.pallas_call(
        _tc2_body,
        grid=(N // BR,),
        in_specs=[
            pl.BlockSpec((NC, BR, D_HID // 2), lambda i: (0, i, 0)),
            _degp_spec,
            pl.BlockSpec((1, D_HID), lambda i: (0, 0)),
            pl.BlockSpec((D_HID, D_OUT), lambda i: (0, 0)),
        ],
        out_specs=pl.BlockSpec((NC, BR, D_OUT // 2), lambda i: (0, i, 0)),
        out_shape=jax.ShapeDtypeStruct((NC, N, D_OUT // 2), jnp.float32),
    )(agg1, degp, b1.reshape(1, D_HID), W2)


def _tc_final(agg2, degp, b2):
    return pl.pallas_call(
        _tc3_body,
        grid=(N // BR,),
        in_specs=[
            pl.BlockSpec((NC, BR, D_OUT // 2), lambda i: (0, i, 0)),
            _degp_spec,
            pl.BlockSpec((1, D_OUT), lambda i: (0, 0)),
        ],
        out_specs=pl.BlockSpec((BR, D_OUT), lambda i: (i, 0)),
        out_shape=jax.ShapeDtypeStruct((N, D_OUT), jnp.float32),
    )(agg2, degp, b2.reshape(1, D_OUT))


def kernel(x, edge_index, cache_name, W1, b1, W2, b2):
    src = edge_index[0].astype(jnp.int32)
    dst = edge_index[1].astype(jnp.int32)
    pad = EPAD - E
    src_p = jnp.concatenate([src, jnp.zeros((pad,), jnp.int32)])
    dst_p = jnp.concatenate([dst, jnp.full((pad,), N, jnp.int32)])
    const = jnp.concatenate(
        [jnp.ones((1, N, DEGW), jnp.float32), jnp.zeros((1, N, DEGW), jnp.float32)],
        axis=0,
    )

    degp = _sc_degree(dst_p, const)
    hp1 = _tc_linear1(x, W1, degp).reshape(NC * N, D_HID // 2)
    agg1 = _sc_agg_128(hp1, src_p, dst_p)
    hp2 = _tc_linear2(agg1, degp, b1, W2).reshape(NC * N, D_OUT // 2)
    agg2 = _sc_agg_64(hp2, src_p, dst_p)
    return _tc_final(agg2, degp, b2)


# speedup vs baseline: 5.9811x; 5.9811x over previous
import functools

import jax
import jax.numpy as jnp
from jax import lax
from jax.experimental import pallas as pl
from jax.experimental.pallas import tpu as pltpu
from jax.experimental.pallas import tpu_sc as plsc

N = 10000
NP = 10240
E = 160000
D_IN = 256
D_HID = 256
D_OUT = 128

NC = 2
NS = 16
K = 128
EPAD = 163840
RPT = NP // NS
DEGW = 16

_mesh = plsc.VectorSubcoreMesh(core_axis_name="c", subcore_axis_name="s")


@functools.partial(
    pl.kernel,
    out_type=jax.ShapeDtypeStruct((NC, NP, DEGW), jnp.float32),
    mesh=_mesh,
    scratch_types=[
        pltpu.VMEM((K,), jnp.int32),
        pltpu.VMEM((K, DEGW), jnp.float32),
        pltpu.VMEM_SHARED((NP + 8, DEGW), jnp.float32),
    ],
)
def _sc_degree(dst_hbm, const_hbm, out_hbm, idxd, ones_v, acc):
    c = lax.axis_index("c")
    s = lax.axis_index("s")
    rbase = s * RPT
    pltpu.sync_copy(const_hbm.at[c, pl.ds(rbase, RPT)], acc.at[pl.ds(rbase, RPT)])
    pltpu.sync_copy(const_hbm.at[0, pl.ds(0, K)], ones_v)
    plsc.subcore_barrier()

    ept = EPAD // (NC * NS)
    ebase = (c * NS + s) * ept

    def body(i, carry):
        eoff = ebase + i * K
        pltpu.sync_copy(dst_hbm.at[pl.ds(eoff, K)], idxd)
        pltpu.sync_copy(ones_v, acc.at[idxd], add=True)
        return carry

    lax.fori_loop(0, ept // K, body, 0)
    plsc.subcore_barrier()
    pltpu.sync_copy(acc.at[pl.ds(rbase, RPT)], out_hbm.at[c, pl.ds(rbase, RPT)])


def _make_sc_agg(W):
    @functools.partial(
        pl.kernel,
        out_type=jax.ShapeDtypeStruct((NC, NP, W), jnp.float32),
        mesh=_mesh,
        scratch_types=[
            pltpu.VMEM((K,), jnp.int32),
            pltpu.VMEM((K,), jnp.int32),
            pltpu.VMEM((K,), jnp.int32),
            pltpu.VMEM((K, W), jnp.float32),
            pltpu.VMEM_SHARED((NP + 8, W), jnp.float32),
            pltpu.SemaphoreType.DMA,
        ],
    )
    def _sc_agg(hp_hbm, src_hbm, dst_hbm, out_hbm, idxs, idxs2, idxd, rows, acc, sem):
        c = lax.axis_index("c")
        s = lax.axis_index("s")
        rbase = s * RPT
        bias = c * NP
        pltpu.sync_copy(hp_hbm.at[pl.ds(bias + rbase, RPT)], acc.at[pl.ds(rbase, RPT)])
        plsc.subcore_barrier()

        ept = EPAD // NS
        ebase = s * ept

        def body(i, carry):
            eoff = ebase + i * K
            pltpu.sync_copy(src_hbm.at[pl.ds(eoff, K)], idxs)
            pltpu.sync_copy(dst_hbm.at[pl.ds(eoff, K)], idxd)
            for j in range(K // 16):
                sl = pl.ds(j * 16, 16)
                idxs2[sl] = idxs[sl] + bias
            pltpu.async_copy(hp_hbm.at[idxs2], rows, sem).wait()
            pltpu.sync_copy(rows, acc.at[idxd], add=True)
            return carry

        lax.fori_loop(0, ept // K, body, 0)
        plsc.subcore_barrier()
        pltpu.sync_copy(acc.at[pl.ds(rbase, RPT)], out_hbm.at[c, pl.ds(rbase, RPT)])

    return _sc_agg


_sc_agg_128 = _make_sc_agg(D_HID // 2)


@functools.partial(
    pl.kernel,
    out_type=jax.ShapeDtypeStruct((NC, NP, D_OUT), jnp.float32),
    mesh=_mesh,
    scratch_types=[
        pltpu.VMEM((K,), jnp.int32),
        pltpu.VMEM((K,), jnp.int32),
        pltpu.VMEM((K, D_OUT), jnp.float32),
        pltpu.VMEM_SHARED((NP + 8, D_OUT), jnp.float32),
        pltpu.SemaphoreType.DMA,
    ],
)
def _sc_agg_full(hp_hbm, zeros_hbm, src_hbm, dst_hbm, out_hbm, idxs, idxd, rows, acc, sem):
    c = lax.axis_index("c")
    s = lax.axis_index("s")
    rbase = s * RPT

    @pl.when(c == 0)
    def _():
        pltpu.sync_copy(hp_hbm.at[pl.ds(rbase, RPT)], acc.at[pl.ds(rbase, RPT)])

    @pl.when(c == 1)
    def _():
        pltpu.sync_copy(zeros_hbm.at[pl.ds(rbase, RPT)], acc.at[pl.ds(rbase, RPT)])

    plsc.subcore_barrier()

    ept = EPAD // (NC * NS)
    ebase = (c * NS + s) * ept

    def body(i, carry):
        eoff = ebase + i * K
        pltpu.sync_copy(src_hbm.at[pl.ds(eoff, K)], idxs)
        pltpu.sync_copy(dst_hbm.at[pl.ds(eoff, K)], idxd)
        pltpu.async_copy(hp_hbm.at[idxs], rows, sem).wait()
        pltpu.sync_copy(rows, acc.at[idxd], add=True)
        return carry

    lax.fori_loop(0, ept // K, body, 0)
    plsc.subcore_barrier()
    pltpu.sync_copy(acc.at[pl.ds(rbase, RPT)], out_hbm.at[c, pl.ds(rbase, RPT)])


BR = 1000


def _dinv_of(degp):
    return lax.rsqrt(degp[0, :, 0] + degp[1, :, 0])


def _tc1_body(x_ref, w_ref, degp_ref, hp_ref):
    dinv = _dinv_of(degp_ref[...])
    h = jnp.dot(x_ref[...], w_ref[...], preferred_element_type=jnp.float32)
    hp = h * dinv[:, None]
    hp_ref[0] = hp[:, : D_HID // 2]
    hp_ref[1] = hp[:, D_HID // 2 :]


def _tc2_body(agg_ref, degp_ref, b1_ref, w2_ref, hp2_ref):
    a = agg_ref[...]
    dinv = _dinv_of(degp_ref[...])
    h1 = jnp.concatenate([a[0], a[1]], axis=1) * dinv[:, None] + b1_ref[...]
    h1 = jnp.maximum(h1, 0.0)
    h2 = jnp.dot(h1, w2_ref[...], preferred_element_type=jnp.float32)
    hp2_ref[...] = h2 * dinv[:, None]


def _tc3_body(agg_ref, degp_ref, b2_ref, out_ref):
    a = agg_ref[...]
    dinv = _dinv_of(degp_ref[...])
    out = (a[0] + a[1]) * dinv[:, None] + b2_ref[...]
    out_ref[...] = jnp.maximum(out, 0.0)


_degp_spec = pl.BlockSpec((NC, BR, DEGW), lambda i: (0, i, 0))


def _tc_linear1(x, W1, degp):
    return pl.pallas_call(
        _tc1_body,
        grid=(N // BR,),
        in_specs=[
            pl.BlockSpec((BR, D_IN), lambda i: (i, 0)),
            pl.BlockSpec((D_IN, D_HID), lambda i: (0, 0)),
            _degp_spec,
        ],
        out_specs=pl.BlockSpec((NC, BR, D_HID // 2), lambda i: (0, i, 0)),
        out_shape=jax.ShapeDtypeStruct((NC, NP, D_HID // 2), jnp.float32),
    )(x, W1, degp)


def _tc_linear2(agg1, degp, b1, W2):
    return pl.pallas_call(
        _tc2_body,
        grid=(N // BR,),
        in_specs=[
            pl.BlockSpec((NC, BR, D_HID // 2), lambda i: (0, i, 0)),
            _degp_spec,
            pl.BlockSpec((1, D_HID), lambda i: (0, 0)),
            pl.BlockSpec((D_HID, D_OUT), lambda i: (0, 0)),
        ],
        out_specs=pl.BlockSpec((BR, D_OUT), lambda i: (i, 0)),
        out_shape=jax.ShapeDtypeStruct((NP, D_OUT), jnp.float32),
    )(agg1, degp, b1.reshape(1, D_HID), W2)


def _tc_final(agg2, degp, b2):
    return pl.pallas_call(
        _tc3_body,
        grid=(N // BR,),
        in_specs=[
            pl.BlockSpec((NC, BR, D_OUT), lambda i: (0, i, 0)),
            _degp_spec,
            pl.BlockSpec((1, D_OUT), lambda i: (0, 0)),
        ],
        out_specs=pl.BlockSpec((BR, D_OUT), lambda i: (i, 0)),
        out_shape=jax.ShapeDtypeStruct((N, D_OUT), jnp.float32),
    )(agg2, degp, b2.reshape(1, D_OUT))


def kernel(x, edge_index, cache_name, W1, b1, W2, b2):
    src = edge_index[0].astype(jnp.int32)
    dst = edge_index[1].astype(jnp.int32)
    pad = EPAD - E
    src_p = jnp.concatenate([src, jnp.zeros((pad,), jnp.int32)])
    dst_p = jnp.concatenate([dst, jnp.full((pad,), NP, jnp.int32)])
    const = jnp.concatenate(
        [jnp.ones((1, NP, DEGW), jnp.float32), jnp.zeros((1, NP, DEGW), jnp.float32)],
        axis=0,
    )

    zeros_np = jnp.zeros((NP, D_OUT), jnp.float32)

    degp = _sc_degree(dst_p, const)
    hp1 = _tc_linear1(x, W1, degp).reshape(NC * NP, D_HID // 2)
    agg1 = _sc_agg_128(hp1, src_p, dst_p)
    hp2 = _tc_linear2(agg1, degp, b1, W2)
    agg2 = _sc_agg_full(hp2, zeros_np, src_p, dst_p)
    return _tc_final(agg2, degp, b2)


# speedup vs baseline: 8.8117x; 1.4733x over previous
import functools

import jax
import jax.numpy as jnp
from jax import lax
from jax.experimental import pallas as pl
from jax.experimental.pallas import tpu as pltpu
from jax.experimental.pallas import tpu_sc as plsc

N = 10000
NP = 10240
E = 160000
D_IN = 256
D_HID = 256
D_OUT = 128

NC = 2
NS = 16
K = 128
EPAD = 163840
RPT = NP // NS
DEGW = 16

_mesh = plsc.VectorSubcoreMesh(core_axis_name="c", subcore_axis_name="s")


NCH_HALF = EPAD // (NC * NS * K)
NCH_FULL = EPAD // (NS * K)
NB = 2
SB = 16


@functools.partial(
    pl.kernel,
    out_type=jax.ShapeDtypeStruct((NC, NP, DEGW), jnp.float32),
    mesh=_mesh,
    scratch_types=[
        pltpu.VMEM((NCH_HALF, K), jnp.int32),
        pltpu.VMEM((K, DEGW), jnp.float32),
        pltpu.VMEM_SHARED((NP + 8, DEGW), jnp.float32),
        pltpu.SemaphoreType.DMA,
    ],
)
def _sc_degree(dst_hbm, const_hbm, out_hbm, idxd, ones_v, acc, sem):
    c = lax.axis_index("c")
    s = lax.axis_index("s")
    rbase = s * RPT
    cbase = (c * NS + s) * NCH_HALF
    pltpu.sync_copy(dst_hbm.at[pl.ds(cbase, NCH_HALF)], idxd)
    pltpu.sync_copy(const_hbm.at[c, pl.ds(rbase, RPT)], acc.at[pl.ds(rbase, RPT)])
    pltpu.sync_copy(const_hbm.at[0, pl.ds(0, K)], ones_v)
    plsc.subcore_barrier()

    for i in range(NCH_HALF):
        pltpu.async_copy(ones_v, acc.at[idxd.at[i]], sem, add=True)
    for i in range(NCH_HALF):
        pltpu.make_async_copy(ones_v, acc.at[idxd.at[i]], sem).wait()
    plsc.subcore_barrier()
    pltpu.sync_copy(acc.at[pl.ds(rbase, RPT)], out_hbm.at[c, pl.ds(rbase, RPT)])


def _agg_pipeline(hp_hbm, idx_pairs, rows, acc, gsem, ssem):
    nch = len(idx_pairs)
    for b in range(NB):
        pltpu.async_copy(hp_hbm.at[idx_pairs[b][0]], rows.at[b], gsem.at[b])
    for i in range(nch):
        b = i % NB
        si, di = idx_pairs[i]
        pltpu.make_async_copy(hp_hbm.at[si], rows.at[b], gsem.at[b]).wait()
        pltpu.async_copy(rows.at[b], acc.at[di], ssem.at[b], add=True)
        pltpu.make_async_copy(rows.at[b], acc.at[di], ssem.at[b]).wait()
        if i + NB < nch:
            pltpu.async_copy(hp_hbm.at[idx_pairs[i + NB][0]], rows.at[b], gsem.at[b])


HW = D_HID // 2
NSB = NCH_FULL // SB

@functools.partial(
    pl.kernel,
    out_type=jax.ShapeDtypeStruct((NC, NP, HW), jnp.float32),
    mesh=_mesh,
    scratch_types=[
        pltpu.VMEM((2, SB, K), jnp.int32),
        pltpu.VMEM((NCH_FULL, K), jnp.int32),
        pltpu.VMEM((NB, K, HW), jnp.float32),
        pltpu.VMEM_SHARED((NP + 8, HW), jnp.float32),
        pltpu.SemaphoreType.DMA((2,)),
        pltpu.SemaphoreType.DMA((NB,)),
        pltpu.SemaphoreType.DMA((NB,)),
    ],
)
def _sc_agg_128(hp_hbm, src2d_hbm, dst2d_hbm, out_hbm, idxs, idxd, rows, acc, isem, gsem, ssem):
    c = lax.axis_index("c")
    s = lax.axis_index("s")
    rbase = s * RPT
    bias = c * NP
    cbase = s * NCH_FULL
    pltpu.sync_copy(dst2d_hbm.at[pl.ds(cbase, NCH_FULL)], idxd)
    for sl in range(min(2, NSB)):
        pltpu.async_copy(src2d_hbm.at[pl.ds(cbase + sl * SB, SB)], idxs.at[sl], isem.at[sl])
    pltpu.sync_copy(hp_hbm.at[pl.ds(bias + rbase, RPT)], acc.at[pl.ds(rbase, RPT)])
    plsc.subcore_barrier()

    for sb in range(NSB):
        sl = sb % 2
        pltpu.make_async_copy(
            src2d_hbm.at[pl.ds(cbase + sb * SB, SB)], idxs.at[sl], isem.at[sl]
        ).wait()
        for r in range(SB):
            for j in range(K // 16):
                ds16 = pl.ds(j * 16, 16)
                idxs[sl, r, ds16] = idxs[sl, r, ds16] + bias
        pairs = [(idxs.at[sl, r], idxd.at[sb * SB + r]) for r in range(SB)]
        _agg_pipeline(hp_hbm, pairs, rows, acc, gsem, ssem)
        if sb + 2 < NSB:
            pltpu.async_copy(
                src2d_hbm.at[pl.ds(cbase + (sb + 2) * SB, SB)], idxs.at[sl], isem.at[sl]
            )

    plsc.subcore_barrier()
    pltpu.sync_copy(acc.at[pl.ds(rbase, RPT)], out_hbm.at[c, pl.ds(rbase, RPT)])


@functools.partial(
    pl.kernel,
    out_type=jax.ShapeDtypeStruct((NC, NP, D_OUT), jnp.float32),
    mesh=_mesh,
    scratch_types=[
        pltpu.VMEM((NCH_HALF, K), jnp.int32),
        pltpu.VMEM((NCH_HALF, K), jnp.int32),
        pltpu.VMEM((NB, K, D_OUT), jnp.float32),
        pltpu.VMEM_SHARED((NP + 8, D_OUT), jnp.float32),
        pltpu.SemaphoreType.DMA((NB,)),
        pltpu.SemaphoreType.DMA((NB,)),
    ],
)
def _sc_agg_full(hp_hbm, zeros_hbm, src2d_hbm, dst2d_hbm, out_hbm, idxs, idxd, rows, acc, gsem, ssem):
    c = lax.axis_index("c")
    s = lax.axis_index("s")
    rbase = s * RPT
    cbase = (c * NS + s) * NCH_HALF
    pltpu.sync_copy(src2d_hbm.at[pl.ds(cbase, NCH_HALF)], idxs)
    pltpu.sync_copy(dst2d_hbm.at[pl.ds(cbase, NCH_HALF)], idxd)

    @pl.when(c == 0)
    def _():
        pltpu.sync_copy(hp_hbm.at[pl.ds(rbase, RPT)], acc.at[pl.ds(rbase, RPT)])

    @pl.when(c == 1)
    def _():
        pltpu.sync_copy(zeros_hbm.at[pl.ds(rbase, RPT)], acc.at[pl.ds(rbase, RPT)])

    plsc.subcore_barrier()
    pairs = [(idxs.at[i], idxd.at[i]) for i in range(NCH_HALF)]
    _agg_pipeline(hp_hbm, pairs, rows, acc, gsem, ssem)
    plsc.subcore_barrier()
    pltpu.sync_copy(acc.at[pl.ds(rbase, RPT)], out_hbm.at[c, pl.ds(rbase, RPT)])


BR = 1000


def _dinv_of(degp):
    return lax.rsqrt(degp[0, :, 0] + degp[1, :, 0])


def _tc1_body(x_ref, w_ref, degp_ref, hp_ref):
    dinv = _dinv_of(degp_ref[...])
    h = jnp.dot(x_ref[...], w_ref[...], preferred_element_type=jnp.float32)
    hp = h * dinv[:, None]
    hp_ref[0] = hp[:, : D_HID // 2]
    hp_ref[1] = hp[:, D_HID // 2 :]


def _tc2_body(agg_ref, degp_ref, b1_ref, w2_ref, hp2_ref):
    a = agg_ref[...]
    dinv = _dinv_of(degp_ref[...])
    h1 = jnp.concatenate([a[0], a[1]], axis=1) * dinv[:, None] + b1_ref[...]
    h1 = jnp.maximum(h1, 0.0)
    h2 = jnp.dot(h1, w2_ref[...], preferred_element_type=jnp.float32)
    hp2_ref[...] = h2 * dinv[:, None]


def _tc3_body(agg_ref, degp_ref, b2_ref, out_ref):
    a = agg_ref[...]
    dinv = _dinv_of(degp_ref[...])
    out = (a[0] + a[1]) * dinv[:, None] + b2_ref[...]
    out_ref[...] = jnp.maximum(out, 0.0)


_degp_spec = pl.BlockSpec((NC, BR, DEGW), lambda i: (0, i, 0))


def _tc_linear1(x, W1, degp):
    return pl.pallas_call(
        _tc1_body,
        grid=(N // BR,),
        in_specs=[
            pl.BlockSpec((BR, D_IN), lambda i: (i, 0)),
            pl.BlockSpec((D_IN, D_HID), lambda i: (0, 0)),
            _degp_spec,
        ],
        out_specs=pl.BlockSpec((NC, BR, D_HID // 2), lambda i: (0, i, 0)),
        out_shape=jax.ShapeDtypeStruct((NC, NP, D_HID // 2), jnp.float32),
    )(x, W1, degp)


def _tc_linear2(agg1, degp, b1, W2):
    return pl.pallas_call(
        _tc2_body,
        grid=(N // BR,),
        in_specs=[
            pl.BlockSpec((NC, BR, D_HID // 2), lambda i: (0, i, 0)),
            _degp_spec,
            pl.BlockSpec((1, D_HID), lambda i: (0, 0)),
            pl.BlockSpec((D_HID, D_OUT), lambda i: (0, 0)),
        ],
        out_specs=pl.BlockSpec((BR, D_OUT), lambda i: (i, 0)),
        out_shape=jax.ShapeDtypeStruct((NP, D_OUT), jnp.float32),
    )(agg1, degp, b1.reshape(1, D_HID), W2)


def _tc_final(agg2, degp, b2):
    return pl.pallas_call(
        _tc3_body,
        grid=(N // BR,),
        in_specs=[
            pl.BlockSpec((NC, BR, D_OUT), lambda i: (0, i, 0)),
            _degp_spec,
            pl.BlockSpec((1, D_OUT), lambda i: (0, 0)),
        ],
        out_specs=pl.BlockSpec((BR, D_OUT), lambda i: (i, 0)),
        out_shape=jax.ShapeDtypeStruct((N, D_OUT), jnp.float32),
    )(agg2, degp, b2.reshape(1, D_OUT))


def kernel(x, edge_index, cache_name, W1, b1, W2, b2):
    src = edge_index[0].astype(jnp.int32)
    dst = edge_index[1].astype(jnp.int32)
    pad = EPAD - E
    src_p = jnp.concatenate([src, jnp.zeros((pad,), jnp.int32)])
    dst_p = jnp.concatenate([dst, jnp.full((pad,), NP, jnp.int32)])
    const = jnp.concatenate(
        [jnp.ones((1, NP, DEGW), jnp.float32), jnp.zeros((1, NP, DEGW), jnp.float32)],
        axis=0,
    )

    zeros_np = jnp.zeros((NP, D_OUT), jnp.float32)
    src2d = src_p.reshape(EPAD // K, K)
    dst2d = dst_p.reshape(EPAD // K, K)

    degp = _sc_degree(dst2d, const)
    hp1 = _tc_linear1(x, W1, degp).reshape(NC * NP, D_HID // 2)
    agg1 = _sc_agg_128(hp1, src2d, dst2d)
    hp2 = _tc_linear2(agg1, degp, b1, W2)
    agg2 = _sc_agg_full(hp2, zeros_np, src2d, dst2d)
    return _tc_final(agg2, degp, b2)


# speedup vs baseline: 8.8702x; 1.0066x over previous
import functools

import jax
import jax.numpy as jnp
from jax import lax
from jax.experimental import pallas as pl
from jax.experimental.pallas import tpu as pltpu
from jax.experimental.pallas import tpu_sc as plsc

N = 10000
NP = 10240
E = 160000
D_IN = 256
D_HID = 256
D_OUT = 128

NC = 2
NS = 16
K = 128
EPAD = 163840
RPT = NP // NS
DEGW = 16

_mesh = plsc.VectorSubcoreMesh(core_axis_name="c", subcore_axis_name="s")


NCH_HALF = EPAD // (NC * NS * K)
NCH_FULL = EPAD // (NS * K)
NB = 2
SB = 16


@functools.partial(
    pl.kernel,
    out_type=jax.ShapeDtypeStruct((NC, NP, DEGW), jnp.float32),
    mesh=_mesh,
    scratch_types=[
        pltpu.VMEM((NCH_HALF, K), jnp.int32),
        pltpu.VMEM((K, DEGW), jnp.float32),
        pltpu.VMEM_SHARED((NP + K, DEGW), jnp.float32),
        pltpu.SemaphoreType.DMA,
    ],
)
def _sc_degree(dst_hbm, const_hbm, out_hbm, idxd, ones_v, acc, sem):
    c = lax.axis_index("c")
    s = lax.axis_index("s")
    rbase = s * RPT
    cbase = (c * NS + s) * NCH_HALF
    pltpu.sync_copy(dst_hbm.at[pl.ds(cbase, NCH_HALF)], idxd)
    pltpu.sync_copy(const_hbm.at[c, pl.ds(rbase, RPT)], acc.at[pl.ds(rbase, RPT)])
    pltpu.sync_copy(const_hbm.at[0, pl.ds(0, K)], ones_v)
    plsc.subcore_barrier()

    for i in range(NCH_HALF):
        pltpu.async_copy(ones_v, acc.at[idxd.at[i]], sem, add=True)
    for i in range(NCH_HALF):
        pltpu.make_async_copy(ones_v, acc.at[idxd.at[i]], sem).wait()
    plsc.subcore_barrier()
    pltpu.sync_copy(acc.at[pl.ds(rbase, RPT)], out_hbm.at[c, pl.ds(rbase, RPT)])


def _agg_pipeline(hp_hbm, idx_pairs, rows, acc, gsem, ssem):
    nch = len(idx_pairs)
    for b in range(NB):
        pltpu.async_copy(hp_hbm.at[idx_pairs[b][0]], rows.at[b], gsem.at[b])
    for i in range(nch):
        b = i % NB
        si, di = idx_pairs[i]
        pltpu.make_async_copy(hp_hbm.at[si], rows.at[b], gsem.at[b]).wait()
        pltpu.async_copy(rows.at[b], acc.at[di], ssem.at[b], add=True)
        pltpu.make_async_copy(rows.at[b], acc.at[di], ssem.at[b]).wait()
        if i + NB < nch:
            pltpu.async_copy(hp_hbm.at[idx_pairs[i + NB][0]], rows.at[b], gsem.at[b])


HW = D_HID // 2
NSB = NCH_FULL // SB

@functools.partial(
    pl.kernel,
    out_type=jax.ShapeDtypeStruct((NC, NP, HW), jnp.float32),
    mesh=_mesh,
    scratch_types=[
        pltpu.VMEM((2, SB, K), jnp.int32),
        pltpu.VMEM((NCH_FULL, K), jnp.int32),
        pltpu.VMEM((NB, K, HW), jnp.float32),
        pltpu.VMEM_SHARED((NP + K, HW), jnp.float32),
        pltpu.SemaphoreType.DMA((2,)),
        pltpu.SemaphoreType.DMA((NB,)),
        pltpu.SemaphoreType.DMA((NB,)),
    ],
)
def _sc_agg_128(hp_hbm, src2d_hbm, dst2d_hbm, out_hbm, idxs, idxd, rows, acc, isem, gsem, ssem):
    c = lax.axis_index("c")
    s = lax.axis_index("s")
    rbase = s * RPT
    bias = c * NP
    cbase = s * NCH_FULL
    pltpu.sync_copy(dst2d_hbm.at[pl.ds(cbase, NCH_FULL)], idxd)
    for sl in range(min(2, NSB)):
        pltpu.async_copy(src2d_hbm.at[pl.ds(cbase + sl * SB, SB)], idxs.at[sl], isem.at[sl])
    pltpu.sync_copy(hp_hbm.at[pl.ds(bias + rbase, RPT)], acc.at[pl.ds(rbase, RPT)])
    plsc.subcore_barrier()

    for sb in range(NSB):
        sl = sb % 2
        pltpu.make_async_copy(
            src2d_hbm.at[pl.ds(cbase + sb * SB, SB)], idxs.at[sl], isem.at[sl]
        ).wait()
        for r in range(SB):
            for j in range(K // 16):
                ds16 = pl.ds(j * 16, 16)
                idxs[sl, r, ds16] = idxs[sl, r, ds16] + bias
        pairs = [(idxs.at[sl, r], idxd.at[sb * SB + r]) for r in range(SB)]
        _agg_pipeline(hp_hbm, pairs, rows, acc, gsem, ssem)
        if sb + 2 < NSB:
            pltpu.async_copy(
                src2d_hbm.at[pl.ds(cbase + (sb + 2) * SB, SB)], idxs.at[sl], isem.at[sl]
            )

    plsc.subcore_barrier()
    pltpu.sync_copy(acc.at[pl.ds(rbase, RPT)], out_hbm.at[c, pl.ds(rbase, RPT)])


@functools.partial(
    pl.kernel,
    out_type=jax.ShapeDtypeStruct((NC, NP, D_OUT), jnp.float32),
    mesh=_mesh,
    scratch_types=[
        pltpu.VMEM((NCH_HALF, K), jnp.int32),
        pltpu.VMEM((NCH_HALF, K), jnp.int32),
        pltpu.VMEM((NB, K, D_OUT), jnp.float32),
        pltpu.VMEM_SHARED((NP + K, D_OUT), jnp.float32),
        pltpu.SemaphoreType.DMA((NB,)),
        pltpu.SemaphoreType.DMA((NB,)),
    ],
)
def _sc_agg_full(hp_hbm, zeros_hbm, src2d_hbm, dst2d_hbm, out_hbm, idxs, idxd, rows, acc, gsem, ssem):
    c = lax.axis_index("c")
    s = lax.axis_index("s")
    rbase = s * RPT
    cbase = (c * NS + s) * NCH_HALF
    pltpu.sync_copy(src2d_hbm.at[pl.ds(cbase, NCH_HALF)], idxs)
    pltpu.sync_copy(dst2d_hbm.at[pl.ds(cbase, NCH_HALF)], idxd)

    @pl.when(c == 0)
    def _():
        pltpu.sync_copy(hp_hbm.at[pl.ds(rbase, RPT)], acc.at[pl.ds(rbase, RPT)])

    @pl.when(c == 1)
    def _():
        pltpu.sync_copy(zeros_hbm.at[pl.ds(rbase, RPT)], acc.at[pl.ds(rbase, RPT)])

    plsc.subcore_barrier()
    pairs = [(idxs.at[i], idxd.at[i]) for i in range(NCH_HALF)]
    _agg_pipeline(hp_hbm, pairs, rows, acc, gsem, ssem)
    plsc.subcore_barrier()
    pltpu.sync_copy(acc.at[pl.ds(rbase, RPT)], out_hbm.at[c, pl.ds(rbase, RPT)])


BR = 1000


def _dinv_of(degp):
    return lax.rsqrt(degp[0, :, 0] + degp[1, :, 0])


def _tc1_body(x_ref, w_ref, degp_ref, hp_ref):
    dinv = _dinv_of(degp_ref[...])
    h = jnp.dot(x_ref[...], w_ref[...], preferred_element_type=jnp.float32)
    hp = h * dinv[:, None]
    hp_ref[0] = hp[:, : D_HID // 2]
    hp_ref[1] = hp[:, D_HID // 2 :]


def _tc2_body(agg_ref, degp_ref, b1_ref, w2_ref, hp2_ref):
    a = agg_ref[...]
    dinv = _dinv_of(degp_ref[...])
    h1 = jnp.concatenate([a[0], a[1]], axis=1) * dinv[:, None] + b1_ref[...]
    h1 = jnp.maximum(h1, 0.0)
    h2 = jnp.dot(h1, w2_ref[...], preferred_element_type=jnp.float32)
    hp2_ref[...] = h2 * dinv[:, None]


def _tc3_body(agg_ref, degp_ref, b2_ref, out_ref):
    a = agg_ref[...]
    dinv = _dinv_of(degp_ref[...])
    out = (a[0] + a[1]) * dinv[:, None] + b2_ref[...]
    out_ref[...] = jnp.maximum(out, 0.0)


_degp_spec = pl.BlockSpec((NC, BR, DEGW), lambda i: (0, i, 0))


def _tc_linear1(x, W1, degp):
    return pl.pallas_call(
        _tc1_body,
        grid=(N // BR,),
        in_specs=[
            pl.BlockSpec((BR, D_IN), lambda i: (i, 0)),
            pl.BlockSpec((D_IN, D_HID), lambda i: (0, 0)),
            _degp_spec,
        ],
        out_specs=pl.BlockSpec((NC, BR, D_HID // 2), lambda i: (0, i, 0)),
        out_shape=jax.ShapeDtypeStruct((NC, NP, D_HID // 2), jnp.float32),
    )(x, W1, degp)


def _tc_linear2(agg1, degp, b1, W2):
    return pl.pallas_call(
        _tc2_body,
        grid=(N // BR,),
        in_specs=[
            pl.BlockSpec((NC, BR, D_HID // 2), lambda i: (0, i, 0)),
            _degp_spec,
            pl.BlockSpec((1, D_HID), lambda i: (0, 0)),
            pl.BlockSpec((D_HID, D_OUT), lambda i: (0, 0)),
        ],
        out_specs=pl.BlockSpec((BR, D_OUT), lambda i: (i, 0)),
        out_shape=jax.ShapeDtypeStruct((NP, D_OUT), jnp.float32),
    )(agg1, degp, b1.reshape(1, D_HID), W2)


def _tc_final(agg2, degp, b2):
    return pl.pallas_call(
        _tc3_body,
        grid=(N // BR,),
        in_specs=[
            pl.BlockSpec((NC, BR, D_OUT), lambda i: (0, i, 0)),
            _degp_spec,
            pl.BlockSpec((1, D_OUT), lambda i: (0, 0)),
        ],
        out_specs=pl.BlockSpec((BR, D_OUT), lambda i: (i, 0)),
        out_shape=jax.ShapeDtypeStruct((N, D_OUT), jnp.float32),
    )(agg2, degp, b2.reshape(1, D_OUT))


def kernel(x, edge_index, cache_name, W1, b1, W2, b2):
    src = edge_index[0].astype(jnp.int32)
    dst = edge_index[1].astype(jnp.int32)
    pad = EPAD - E
    src_p = jnp.concatenate([src, jnp.zeros((pad,), jnp.int32)])
    spill = NP + (jnp.arange(pad, dtype=jnp.int32) % K)
    dst_p = jnp.concatenate([dst, spill])
    const = jnp.concatenate(
        [jnp.ones((1, NP, DEGW), jnp.float32), jnp.zeros((1, NP, DEGW), jnp.float32)],
        axis=0,
    )

    zeros_np = jnp.zeros((NP, D_OUT), jnp.float32)
    src2d = src_p.reshape(EPAD // K, K)
    dst2d = dst_p.reshape(EPAD // K, K)

    degp = _sc_degree(dst2d, const)
    hp1 = _tc_linear1(x, W1, degp).reshape(NC * NP, D_HID // 2)
    agg1 = _sc_agg_128(hp1, src2d, dst2d)
    hp2 = _tc_linear2(agg1, degp, b1, W2)
    agg2 = _sc_agg_full(hp2, zeros_np, src2d, dst2d)
    return _tc_final(agg2, degp, b2)


# speedup vs baseline: 9.9300x; 1.1195x over previous
import functools

import jax
import jax.numpy as jnp
from jax import lax
from jax.experimental import pallas as pl
from jax.experimental.pallas import tpu as pltpu
from jax.experimental.pallas import tpu_sc as plsc

N = 10000
NP = 10240
E = 160000
D_IN = 256
D_HID = 256
D_OUT = 128

NC = 2
NS = 16
K = 64
EPAD = 163840
RPT = NP // NS
DEGW = 16

_mesh = plsc.VectorSubcoreMesh(core_axis_name="c", subcore_axis_name="s")


NCH_HALF = EPAD // (NC * NS * K)
NCH_FULL = EPAD // (NS * K)
NB = 4
LEAD = 2
SB = 16


@functools.partial(
    pl.kernel,
    out_type=jax.ShapeDtypeStruct((NC, NP, DEGW), jnp.float32),
    mesh=_mesh,
    scratch_types=[
        pltpu.VMEM((NCH_HALF, K), jnp.int32),
        pltpu.VMEM((K, DEGW), jnp.float32),
        pltpu.VMEM_SHARED((NP + K, DEGW), jnp.float32),
        pltpu.SemaphoreType.DMA,
    ],
)
def _sc_degree(dst_hbm, const_hbm, out_hbm, idxd, ones_v, acc, sem):
    c = lax.axis_index("c")
    s = lax.axis_index("s")
    rbase = s * RPT
    cbase = (c * NS + s) * NCH_HALF
    pltpu.sync_copy(dst_hbm.at[pl.ds(cbase, NCH_HALF)], idxd)
    pltpu.sync_copy(const_hbm.at[c, pl.ds(rbase, RPT)], acc.at[pl.ds(rbase, RPT)])
    pltpu.sync_copy(const_hbm.at[0, pl.ds(0, K)], ones_v)
    plsc.subcore_barrier()

    for i in range(NCH_HALF):
        pltpu.async_copy(ones_v, acc.at[idxd.at[i]], sem, add=True)
    for i in range(NCH_HALF):
        pltpu.make_async_copy(ones_v, acc.at[idxd.at[i]], sem).wait()
    plsc.subcore_barrier()
    pltpu.sync_copy(acc.at[pl.ds(rbase, RPT)], out_hbm.at[c, pl.ds(rbase, RPT)])


IMMEDIATE_DRAIN = True


def _agg_pipeline(hp_hbm, idx_pairs, rows, acc, gsem, ssem,
                  pre_arm=None, on_gather_done=None, on_scatter_drained=None):
    nch = len(idx_pairs)
    if IMMEDIATE_DRAIN:
        for j in range(NB):
            if pre_arm is not None:
                pre_arm(j)
            pltpu.async_copy(hp_hbm.at[idx_pairs[j][0]], rows.at[j % NB], gsem.at[j % NB])
        for i in range(nch):
            b = i % NB
            pltpu.make_async_copy(hp_hbm.at[idx_pairs[i][0]], rows.at[b], gsem.at[b]).wait()
            if on_gather_done is not None:
                on_gather_done(i)
            pltpu.async_copy(rows.at[b], acc.at[idx_pairs[i][1]], ssem.at[b], add=True)
            pltpu.make_async_copy(rows.at[b], acc.at[idx_pairs[i][1]], ssem.at[b]).wait()
            if on_scatter_drained is not None:
                on_scatter_drained(i)
            j = i + NB
            if j < nch:
                if pre_arm is not None:
                    pre_arm(j)
                pltpu.async_copy(hp_hbm.at[idx_pairs[j][0]], rows.at[b], gsem.at[b])
        return
    for j in range(LEAD):
        if pre_arm is not None:
            pre_arm(j)
        pltpu.async_copy(hp_hbm.at[idx_pairs[j][0]], rows.at[j % NB], gsem.at[j % NB])
    for i in range(nch):
        b = i % NB
        j = i + LEAD
        if j < nch:
            bj = j % NB
            jprev = j - NB
            if jprev >= 0:
                pltpu.make_async_copy(
                    rows.at[bj], acc.at[idx_pairs[jprev][1]], ssem.at[bj]
                ).wait()
                if on_scatter_drained is not None:
                    on_scatter_drained(jprev)
            if pre_arm is not None:
                pre_arm(j)
            pltpu.async_copy(hp_hbm.at[idx_pairs[j][0]], rows.at[bj], gsem.at[bj])
        pltpu.make_async_copy(hp_hbm.at[idx_pairs[i][0]], rows.at[b], gsem.at[b]).wait()
        if on_gather_done is not None:
            on_gather_done(i)
        pltpu.async_copy(rows.at[b], acc.at[idx_pairs[i][1]], ssem.at[b], add=True)
    for i in range(max(0, nch - NB), nch):
        b = i % NB
        pltpu.make_async_copy(rows.at[b], acc.at[idx_pairs[i][1]], ssem.at[b]).wait()


def _run_agg(hp_hbm, src2d_hbm, dst2d_hbm, idxs, idxd, rows, acc,
             isems, isemd, gsem, ssem, cbase, nch, bias):
    nsup = nch // SB
    for sl in range(min(2, nsup)):
        pltpu.async_copy(src2d_hbm.at[pl.ds(cbase + sl * SB, SB)], idxs.at[sl], isems.at[sl])
        pltpu.async_copy(dst2d_hbm.at[pl.ds(cbase + sl * SB, SB)], idxd.at[sl], isemd.at[sl])

    def pre_arm(j):
        if j % SB == 0:
            sup = j // SB
            sl = sup % 2
            pltpu.make_async_copy(
                src2d_hbm.at[pl.ds(cbase + sup * SB, SB)], idxs.at[sl], isems.at[sl]
            ).wait()
            pltpu.make_async_copy(
                dst2d_hbm.at[pl.ds(cbase + sup * SB, SB)], idxd.at[sl], isemd.at[sl]
            ).wait()
            if bias is not None:
                for r in range(SB):
                    for jj in range(K // 16):
                        ds16 = pl.ds(jj * 16, 16)
                        idxs[sl, r, ds16] = idxs[sl, r, ds16] + bias

    def on_gather_done(i):
        if (i + 1) % SB == 0:
            sup = i // SB
            if sup + 2 < nsup:
                pltpu.async_copy(
                    src2d_hbm.at[pl.ds(cbase + (sup + 2) * SB, SB)],
                    idxs.at[sup % 2], isems.at[sup % 2],
                )

    def on_scatter_drained(m):
        if (m + 1) % SB == 0:
            sup = m // SB
            if sup + 2 < nsup:
                pltpu.async_copy(
                    dst2d_hbm.at[pl.ds(cbase + (sup + 2) * SB, SB)],
                    idxd.at[sup % 2], isemd.at[sup % 2],
                )

    pairs = [
        (idxs.at[(i // SB) % 2, i % SB], idxd.at[(i // SB) % 2, i % SB])
        for i in range(nch)
    ]
    _agg_pipeline(hp_hbm, pairs, rows, acc, gsem, ssem,
                  pre_arm, on_gather_done, on_scatter_drained)


HW = D_HID // 2

@functools.partial(
    pl.kernel,
    out_type=jax.ShapeDtypeStruct((NC, NP, HW), jnp.float32),
    mesh=_mesh,
    scratch_types=[
        pltpu.VMEM((2, SB, K), jnp.int32),
        pltpu.VMEM((2, SB, K), jnp.int32),
        pltpu.VMEM((NB, K, HW), jnp.float32),
        pltpu.VMEM_SHARED((NP + K, HW), jnp.float32),
        pltpu.SemaphoreType.DMA((2,)),
        pltpu.SemaphoreType.DMA((2,)),
        pltpu.SemaphoreType.DMA((NB,)),
        pltpu.SemaphoreType.DMA((NB,)),
    ],
)
def _sc_agg_128(hp_hbm, src2d_hbm, dst2d_hbm, out_hbm, idxs, idxd, rows, acc,
                isems, isemd, gsem, ssem):
    c = lax.axis_index("c")
    s = lax.axis_index("s")
    rbase = s * RPT
    bias = c * NP
    cbase = s * NCH_FULL
    pltpu.sync_copy(hp_hbm.at[pl.ds(bias + rbase, RPT)], acc.at[pl.ds(rbase, RPT)])
    plsc.subcore_barrier()
    _run_agg(hp_hbm, src2d_hbm, dst2d_hbm, idxs, idxd, rows, acc,
             isems, isemd, gsem, ssem, cbase, NCH_FULL, bias)
    plsc.subcore_barrier()
    pltpu.sync_copy(acc.at[pl.ds(rbase, RPT)], out_hbm.at[c, pl.ds(rbase, RPT)])


@functools.partial(
    pl.kernel,
    out_type=jax.ShapeDtypeStruct((NC, NP, D_OUT), jnp.float32),
    mesh=_mesh,
    scratch_types=[
        pltpu.VMEM((2, SB, K), jnp.int32),
        pltpu.VMEM((2, SB, K), jnp.int32),
        pltpu.VMEM((NB, K, D_OUT), jnp.float32),
        pltpu.VMEM_SHARED((NP + K, D_OUT), jnp.float32),
        pltpu.SemaphoreType.DMA((2,)),
        pltpu.SemaphoreType.DMA((2,)),
        pltpu.SemaphoreType.DMA((NB,)),
        pltpu.SemaphoreType.DMA((NB,)),
    ],
)
def _sc_agg_full(hp_hbm, zeros_hbm, src2d_hbm, dst2d_hbm, out_hbm, idxs, idxd, rows, acc,
                 isems, isemd, gsem, ssem):
    c = lax.axis_index("c")
    s = lax.axis_index("s")
    rbase = s * RPT
    cbase = (c * NS + s) * NCH_HALF

    @pl.when(c == 0)
    def _():
        pltpu.sync_copy(hp_hbm.at[pl.ds(rbase, RPT)], acc.at[pl.ds(rbase, RPT)])

    @pl.when(c == 1)
    def _():
        pltpu.sync_copy(zeros_hbm.at[pl.ds(rbase, RPT)], acc.at[pl.ds(rbase, RPT)])

    plsc.subcore_barrier()
    _run_agg(hp_hbm, src2d_hbm, dst2d_hbm, idxs, idxd, rows, acc,
             isems, isemd, gsem, ssem, cbase, NCH_HALF, None)
    plsc.subcore_barrier()
    pltpu.sync_copy(acc.at[pl.ds(rbase, RPT)], out_hbm.at[c, pl.ds(rbase, RPT)])


BR = 1000


def _dinv_of(degp):
    return lax.rsqrt(degp[0, :, 0] + degp[1, :, 0])


def _tc1_body(x_ref, w_ref, degp_ref, hp_ref):
    dinv = _dinv_of(degp_ref[...])
    h = jnp.dot(x_ref[...], w_ref[...], preferred_element_type=jnp.float32)
    hp = h * dinv[:, None]
    hp_ref[0] = hp[:, : D_HID // 2]
    hp_ref[1] = hp[:, D_HID // 2 :]


def _tc2_body(agg_ref, degp_ref, b1_ref, w2_ref, hp2_ref):
    a = agg_ref[...]
    dinv = _dinv_of(degp_ref[...])
    h1 = jnp.concatenate([a[0], a[1]], axis=1) * dinv[:, None] + b1_ref[...]
    h1 = jnp.maximum(h1, 0.0)
    h2 = jnp.dot(h1, w2_ref[...], preferred_element_type=jnp.float32)
    hp2_ref[...] = h2 * dinv[:, None]


def _tc3_body(agg_ref, degp_ref, b2_ref, out_ref):
    a = agg_ref[...]
    dinv = _dinv_of(degp_ref[...])
    out = (a[0] + a[1]) * dinv[:, None] + b2_ref[...]
    out_ref[...] = jnp.maximum(out, 0.0)


_degp_spec = pl.BlockSpec((NC, BR, DEGW), lambda i: (0, i, 0))


def _tc_linear1(x, W1, degp):
    return pl.pallas_call(
        _tc1_body,
        grid=(N // BR,),
        in_specs=[
            pl.BlockSpec((BR, D_IN), lambda i: (i, 0)),
            pl.BlockSpec((D_IN, D_HID), lambda i: (0, 0)),
            _degp_spec,
        ],
        out_specs=pl.BlockSpec((NC, BR, D_HID // 2), lambda i: (0, i, 0)),
        out_shape=jax.ShapeDtypeStruct((NC, NP, D_HID // 2), jnp.float32),
    )(x, W1, degp)


def _tc_linear2(agg1, degp, b1, W2):
    return pl.pallas_call(
        _tc2_body,
        grid=(N // BR,),
        in_specs=[
            pl.BlockSpec((NC, BR, D_HID // 2), lambda i: (0, i, 0)),
            _degp_spec,
            pl.BlockSpec((1, D_HID), lambda i: (0, 0)),
            pl.BlockSpec((D_HID, D_OUT), lambda i: (0, 0)),
        ],
        out_specs=pl.BlockSpec((BR, D_OUT), lambda i: (i, 0)),
        out_shape=jax.ShapeDtypeStruct((NP, D_OUT), jnp.float32),
    )(agg1, degp, b1.reshape(1, D_HID), W2)


def _tc_final(agg2, degp, b2):
    return pl.pallas_call(
        _tc3_body,
        grid=(N // BR,),
        in_specs=[
            pl.BlockSpec((NC, BR, D_OUT), lambda i: (0, i, 0)),
            _degp_spec,
            pl.BlockSpec((1, D_OUT), lambda i: (0, 0)),
        ],
        out_specs=pl.BlockSpec((BR, D_OUT), lambda i: (i, 0)),
        out_shape=jax.ShapeDtypeStruct((N, D_OUT), jnp.float32),
    )(agg2, degp, b2.reshape(1, D_OUT))


def kernel(x, edge_index, cache_name, W1, b1, W2, b2):
    src = edge_index[0].astype(jnp.int32)
    dst = edge_index[1].astype(jnp.int32)
    pad = EPAD - E
    src_p = jnp.concatenate([src, jnp.zeros((pad,), jnp.int32)])
    spill = NP + (jnp.arange(pad, dtype=jnp.int32) % K)
    dst_p = jnp.concatenate([dst, spill])
    const = jnp.concatenate(
        [jnp.ones((1, NP, DEGW), jnp.float32), jnp.zeros((1, NP, DEGW), jnp.float32)],
        axis=0,
    )

    zeros_np = jnp.zeros((NP, D_OUT), jnp.float32)
    src2d = src_p.reshape(EPAD // K, K)
    dst2d = dst_p.reshape(EPAD // K, K)

    degp = _sc_degree(dst2d, const)
    hp1 = _tc_linear1(x, W1, degp).reshape(NC * NP, D_HID // 2)
    agg1 = _sc_agg_128(hp1, src2d, dst2d)
    hp2 = _tc_linear2(agg1, degp, b1, W2)
    agg2 = _sc_agg_full(hp2, zeros_np, src2d, dst2d)
    return _tc_final(agg2, degp, b2)
